# baseline (speedup 1.0000x reference)
import math

import jax
import jax.numpy as jnp
from jax import lax
from jax.experimental import pallas as pl
from jax.experimental.pallas import tpu as pltpu

_C = 100000
_B = 1024
_SMOOTH = 0.1
_CONF = 1.0 - _SMOOTH
_EPS = _SMOOTH / (_C - 1)
_K0 = _SMOOTH * math.log(_EPS) + _CONF * math.log(_CONF)

_RB = 16
_NBLK = _B // _RB


def _body(p0, p1, p2, p3, tgt_ref, out_ref, acc):
    i = pl.program_id(0)

    @pl.when(i == 0)
    def _init():
        acc[0, 0] = 0.0
        acc[1, 0] = 0.0
        acc[2, 0] = 0.0

    s = 0.0
    for r in (p0, p1, p2, p3):
        s += jnp.sum(r[0:8, 0:128])
    acc[0, 0] += s
    acc[1, 0] += 0.0
    acc[2, 0] += 0.0

    @pl.when(i == _NBLK - 1)
    def _fin():
        total = (
            acc[0, 0] - _EPS * acc[1, 0] - (_CONF - _EPS) * acc[2, 0]
        ) / _B + _K0
        out_ref[...] = jnp.reshape(total, (1, 1))


def kernel(pred, target):
    tgt = target.astype(jnp.int32).reshape(_B, 1)
    out = pl.pallas_call(
        _body,
        grid=(_NBLK // 4,),
        in_specs=[
            pl.BlockSpec((_RB, _C), lambda i: (4 * i + 0, 0)),
            pl.BlockSpec((_RB, _C), lambda i: (4 * i + 1, 0)),
            pl.BlockSpec((_RB, _C), lambda i: (4 * i + 2, 0)),
            pl.BlockSpec((_RB, _C), lambda i: (4 * i + 3, 0)),
            pl.BlockSpec((_RB, 1), lambda i: (i, 0)),
        ],
        out_specs=pl.BlockSpec((1, 1), lambda i: (0, 0)),
        out_shape=jax.ShapeDtypeStruct((1, 1), jnp.float32),
        scratch_shapes=[pltpu.SMEM((3, 1), jnp.float32)],
    )(pred, pred, pred, pred, tgt)
    return out[0, 0]


# speedup vs baseline: 3.0064x; 3.0064x over previous
import math

import jax
import jax.numpy as jnp
from jax import lax
from jax.experimental import pallas as pl
from jax.experimental.pallas import tpu as pltpu

_C = 100000
_B = 1024
_SMOOTH = 0.1
_CONF = 1.0 - _SMOOTH
_EPS = _SMOOTH / (_C - 1)
_K0 = _SMOOTH * math.log(_EPS) + _CONF * math.log(_CONF)

_VB = 2000
_NBLK = _C // _VB


def _body(x_ref, tgt_ref, out_ref, se_acc, sx_acc, pt_acc):
    j = pl.program_id(0)

    @pl.when(j == 0)
    def _init():
        se_acc[...] = jnp.zeros_like(se_acc)
        sx_acc[...] = jnp.zeros_like(sx_acc)
        pt_acc[...] = jnp.zeros_like(pt_acc)

    x = x_ref[...]
    rows = j * _VB + lax.broadcasted_iota(jnp.int32, (_VB, _B), 0)
    hit = rows == tgt_ref[...]
    se_acc[...] += jnp.sum(jnp.exp(x), axis=0, keepdims=True)
    sx_acc[...] += jnp.sum(x, axis=0, keepdims=True)
    pt_acc[...] += jnp.sum(jnp.where(hit, x, 0.0), axis=0, keepdims=True)

    @pl.when(j == _NBLK - 1)
    def _fin():
        lse = jnp.log(se_acc[...])
        total = (
            jnp.sum(lse)
            - _EPS * jnp.sum(sx_acc[...])
            - (_CONF - _EPS) * jnp.sum(pt_acc[...])
        ) / _B + _K0
        out_ref[...] = jnp.reshape(total, (1, 1))


def kernel(pred, target):
    pred_t = pred.T
    tgt = target.astype(jnp.int32).reshape(1, _B)
    out = pl.pallas_call(
        _body,
        grid=(_NBLK,),
        in_specs=[
            pl.BlockSpec((_VB, _B), lambda j: (j, 0)),
            pl.BlockSpec((1, _B), lambda j: (0, 0)),
        ],
        out_specs=pl.BlockSpec((1, 1), lambda j: (0, 0)),
        out_shape=jax.ShapeDtypeStruct((1, 1), jnp.float32),
        scratch_shapes=[
            pltpu.VMEM((1, _B), jnp.float32),
            pltpu.VMEM((1, _B), jnp.float32),
            pltpu.VMEM((1, _B), jnp.float32),
        ],
    )(pred_t, tgt)
    return out[0, 0]


# speedup vs baseline: 3.2417x; 1.0782x over previous
import math

import jax
import jax.numpy as jnp
from jax import lax
from jax.experimental import pallas as pl
from jax.experimental.pallas import tpu as pltpu

_C = 100000
_B = 1024
_SMOOTH = 0.1
_CONF = 1.0 - _SMOOTH
_EPS = _SMOOTH / (_C - 1)
_K0 = _SMOOTH * math.log(_EPS) + _CONF * math.log(_CONF)

_VB = 4000
_NBLK = _C // _VB


def _body(x_ref, tgt_ref, out_ref, se_acc, sx_acc, pt_acc):
    j = pl.program_id(0)

    @pl.when(j == 0)
    def _init():
        se_acc[...] = jnp.zeros_like(se_acc)
        sx_acc[...] = jnp.zeros_like(sx_acc)
        pt_acc[...] = jnp.zeros_like(pt_acc)

    x = x_ref[...]
    rows = j * _VB + lax.broadcasted_iota(jnp.int32, (_VB, _B), 0)
    hit = rows == tgt_ref[...]
    se_acc[...] += jnp.sum(jnp.exp(x), axis=0, keepdims=True)
    sx_acc[...] += jnp.sum(x, axis=0, keepdims=True)
    pt_acc[...] += jnp.sum(jnp.where(hit, x, 0.0), axis=0, keepdims=True)

    @pl.when(j == _NBLK - 1)
    def _fin():
        lse = jnp.log(se_acc[...])
        total = (
            jnp.sum(lse)
            - _EPS * jnp.sum(sx_acc[...])
            - (_CONF - _EPS) * jnp.sum(pt_acc[...])
        ) / _B + _K0
        out_ref[...] = jnp.reshape(total, (1, 1))


def kernel(pred, target):
    pred_t = pred.T
    tgt = target.astype(jnp.int32).reshape(1, _B)
    out = pl.pallas_call(
        _body,
        grid=(_NBLK,),
        in_specs=[
            pl.BlockSpec((_VB, _B), lambda j: (j, 0)),
            pl.BlockSpec((1, _B), lambda j: (0, 0)),
        ],
        out_specs=pl.BlockSpec((1, 1), lambda j: (0, 0)),
        out_shape=jax.ShapeDtypeStruct((1, 1), jnp.float32),
        scratch_shapes=[
            pltpu.VMEM((1, _B), jnp.float32),
            pltpu.VMEM((1, _B), jnp.float32),
            pltpu.VMEM((1, _B), jnp.float32),
        ],
    )(pred_t, tgt)
    return out[0, 0]
